# baseline (speedup 1.0000x reference)
import functools

import jax
import jax.numpy as jnp
from jax import lax
from jax.experimental import pallas as pl
from jax.experimental.pallas import tpu as pltpu
from jax.experimental.pallas import tpu_sc as plsc

_LANES = 16
_BLK = 128
_SUB = 8


def _flatten_table(table):
    v, k = table.shape
    padc = (-v) % _BLK
    nt = (v + padc) // _BLK
    tp = jnp.pad(table.T, ((0, 0), (0, padc)))
    flat = tp.reshape(k // _SUB, _SUB, nt, _BLK)
    flat = flat.transpose(0, 2, 1, 3).reshape(-1)
    return flat, nt


def _flat_indices(ids, nt, k, gidx):
    def calc_idx(t, carry):
        tsl = pl.ds(t * _LANES, _LANES)
        vec = ids[tsl]
        base = ((vec >> 7) << 10) + (vec & 127)
        for kk in range(k):
            off = ((kk // _SUB) * nt << 10) + (kk % _SUB) * _BLK
            gidx[kk, tsl] = base + off
        return carry

    lax.fori_loop(0, ids.shape[0] // _LANES, calc_idx, 0)


def _gather_table(hbm, gidx, vals, sem, k, b_per_w):
    def fire(kk, carry):
        pltpu.async_copy(hbm.at[gidx.at[kk]], vals.at[kk], sem)
        return carry

    lax.fori_loop(0, k, fire, 0)

    def drain(d, carry):
        pltpu.make_async_copy(hbm.at[pl.ds(0, b_per_w)],
                              vals.at[0], sem).wait()
        return carry

    lax.fori_loop(0, k, drain, 0)


def _mesh():
    return plsc.VectorSubcoreMesh(core_axis_name="c", subcore_axis_name="s")


def _compiler_params():
    return pltpu.CompilerParams(
        needs_layout_passes=False, use_tc_tiling_on_sc=False)


def _make_prep_kernel(num_workers, nc, b_per_w, k_dim, nt_i):
    @functools.partial(
        pl.kernel,
        mesh=_mesh(),
        out_type=(
            jax.ShapeDtypeStruct((num_workers, b_per_w), jnp.float32),
            jax.ShapeDtypeStruct((num_workers, b_per_w), jnp.float32),
            jax.ShapeDtypeStruct((num_workers, k_dim, b_per_w), jnp.float32),
        ),
        compiler_params=_compiler_params(),
        scratch_types=[
            pltpu.VMEM((b_per_w,), jnp.int32),
            pltpu.VMEM((b_per_w,), jnp.int32),
            pltpu.VMEM((b_per_w,), jnp.float32),
            pltpu.VMEM((b_per_w,), jnp.float32),
            pltpu.VMEM((k_dim, b_per_w), jnp.int32),
            pltpu.VMEM((k_dim, b_per_w), jnp.float32),
            pltpu.SemaphoreType.DMA,
            pltpu.SemaphoreType.DMA,
        ],
    )
    def prep_kernel(su_hbm, si_hbm, bU_hbm, bI_hbm, gI_hbm, bu_out, bi_out,
                    giv_out, idx_u, idx_i, bu_v, bi_v, gidx_i, gi_v, sem,
                    gsem):
        wid = lax.axis_index("s") * nc + lax.axis_index("c")
        pltpu.sync_copy(su_hbm.at[wid], idx_u)
        pltpu.sync_copy(si_hbm.at[wid], idx_i)
        bu_copy = pltpu.async_copy(bU_hbm.at[idx_u], bu_v, sem)
        bi_copy = pltpu.async_copy(bI_hbm.at[idx_i], bi_v, sem)
        _flat_indices(idx_i, nt_i, k_dim, gidx_i)
        _gather_table(gI_hbm, gidx_i, gi_v, gsem, k_dim, b_per_w)
        bu_copy.wait()
        bi_copy.wait()
        pltpu.sync_copy(bu_v, bu_out.at[wid])
        pltpu.sync_copy(bi_v, bi_out.at[wid])
        pltpu.sync_copy(gi_v, giv_out.at[wid])

    return prep_kernel


def _make_lo_kernel(num_workers, nc, b_per_w, k_lo, nt_u):
    @functools.partial(
        pl.kernel,
        mesh=_mesh(),
        out_type=jax.ShapeDtypeStruct((num_workers, k_lo, b_per_w),
                                      jnp.float32),
        compiler_params=_compiler_params(),
        scratch_types=[
            pltpu.VMEM((b_per_w,), jnp.int32),
            pltpu.VMEM((k_lo, b_per_w), jnp.int32),
            pltpu.VMEM((k_lo, b_per_w), jnp.float32),
            pltpu.SemaphoreType.DMA,
        ],
    )
    def lo_kernel(su_hbm, gUlo_hbm, gul_out, idx_u, gidx_u, gu_v, gsem):
        wid = lax.axis_index("s") * nc + lax.axis_index("c")
        pltpu.sync_copy(su_hbm.at[wid], idx_u)
        _flat_indices(idx_u, nt_u, k_lo, gidx_u)
        _gather_table(gUlo_hbm, gidx_u, gu_v, gsem, k_lo, b_per_w)
        pltpu.sync_copy(gu_v, gul_out.at[wid])

    return lo_kernel


def _make_main_kernel(num_workers, nc, b_per_w, k_dim, k_lo, nt_u):
    n_groups = b_per_w // _LANES
    k_hi = k_dim - k_lo

    @functools.partial(
        pl.kernel,
        mesh=_mesh(),
        out_type=jax.ShapeDtypeStruct((num_workers, _LANES), jnp.float32),
        compiler_params=_compiler_params(),
        scratch_types=[
            pltpu.VMEM((b_per_w,), jnp.int32),
            pltpu.VMEM((b_per_w,), jnp.float32),
            pltpu.VMEM((_LANES,), jnp.float32),
            pltpu.VMEM((b_per_w,), jnp.float32),
            pltpu.VMEM((b_per_w,), jnp.float32),
            pltpu.VMEM((k_hi, b_per_w), jnp.int32),
            pltpu.VMEM((k_lo, b_per_w), jnp.float32),
            pltpu.VMEM((k_hi, b_per_w), jnp.float32),
            pltpu.VMEM((k_dim, b_per_w), jnp.float32),
            pltpu.VMEM((_LANES,), jnp.float32),
            pltpu.SemaphoreType.DMA,
        ],
    )
    def main_kernel(su_hbm, r_hbm, av_hbm, gUhi_hbm, bu_hbm, bi_hbm,
                    gul_hbm, giv_hbm, out_hbm, idx_u, r_v, a_v, bu_v, bi_v,
                    gidx_u, gul_v, guh_v, gi_v, loss_v, gsem):
        wid = lax.axis_index("s") * nc + lax.axis_index("c")
        pltpu.sync_copy(su_hbm.at[wid], idx_u)
        _flat_indices(idx_u, nt_u, k_hi, gidx_u)

        def fire(k, carry):
            pltpu.async_copy(gUhi_hbm.at[gidx_u.at[k]], guh_v.at[k], gsem)
            return carry

        lax.fori_loop(0, k_hi, fire, 0)
        pltpu.sync_copy(r_hbm.at[wid], r_v)
        pltpu.sync_copy(av_hbm, a_v)
        pltpu.sync_copy(bu_hbm.at[wid], bu_v)
        pltpu.sync_copy(bi_hbm.at[wid], bi_v)
        pltpu.sync_copy(gul_hbm.at[wid], gul_v)
        pltpu.sync_copy(giv_hbm.at[wid], gi_v)

        def drain(d, carry):
            pltpu.make_async_copy(gUhi_hbm.at[pl.ds(0, b_per_w)],
                                  guh_v.at[0], gsem).wait()
            return carry

        lax.fori_loop(0, k_hi, drain, 0)
        alpha = a_v[...]

        def group(g, acc_loss):
            sl = pl.ds(g * _LANES, _LANES)
            dot = gul_v[0, sl] * gi_v[0, sl]
            for k in range(1, k_lo):
                dot = dot + gul_v[k, sl] * gi_v[k, sl]
            for k in range(k_hi):
                dot = dot + guh_v[k, sl] * gi_v[k_lo + k, sl]
            pred = alpha + bu_v[sl] + bi_v[sl] + dot
            sig = 5.0 / (1.0 + jnp.exp(-pred))
            diff = sig - r_v[sl]
            return acc_loss + diff * diff

        acc = lax.fori_loop(0, n_groups, group,
                            jnp.zeros((_LANES,), jnp.float32))
        loss_v[...] = acc
        pltpu.sync_copy(loss_v, out_hbm.at[wid])

    return main_kernel


def kernel(sampleU, sampleI, sampleR, alpha, betaU, betaI, gammaU, gammaI):
    info = plsc.get_sparse_core_info()
    nc, ns = info.num_cores, info.num_subcores
    nw = nc * ns
    b = sampleU.shape[0]
    k_dim = gammaU.shape[1]
    k_lo = k_dim // 2
    b_per_w = b // nw
    su = sampleU.astype(jnp.int32).reshape(nw, b_per_w)
    si = sampleI.astype(jnp.int32).reshape(nw, b_per_w)
    r = sampleR.astype(jnp.float32).reshape(nw, b_per_w)
    av = jnp.broadcast_to(jnp.asarray(alpha, jnp.float32), (_LANES,))
    gU_lo, nt_u = _flatten_table(gammaU[:, :k_lo])
    gU_hi, _ = _flatten_table(gammaU[:, k_lo:])
    gI_flat, nt_i = _flatten_table(gammaI)
    prep = _make_prep_kernel(nw, nc, b_per_w, k_dim, nt_i)
    bu, bi, giv = prep(su, si, betaU, betaI, gI_flat)
    lo = _make_lo_kernel(nw, nc, b_per_w, k_lo, nt_u)
    gul = lo(su, gU_lo)
    main = _make_main_kernel(nw, nc, b_per_w, k_dim, k_lo, nt_u)
    out = main(su, r, av, gU_hi, bu, bi, gul, giv)
    return jnp.sum(out) / b
